# Initial kernel scaffold; baseline (speedup 1.0000x reference)
import jax
import jax.numpy as jnp
from jax.experimental import pallas as pl


def kernel(x, pe):
    raise NotImplementedError("write your pallas kernel here")


# speedup vs baseline: 1.9338x; 1.9338x over previous
import functools

import jax
import jax.numpy as jnp
from jax.experimental import pallas as pl


def _add_block(x_ref, pe_ref, o_ref):
    o_ref[...] = x_ref[...] + pe_ref[...][None, :, :]


@functools.partial(jax.jit, static_argnames=("block_s",))
def _pe_add(x, pe, block_s=512):
    b, s, d = x.shape
    grid = (s // block_s, b)
    return pl.pallas_call(
        _add_block,
        grid=grid,
        in_specs=[
            pl.BlockSpec((1, block_s, d), lambda i, j: (j, i, 0)),
            pl.BlockSpec((block_s, d), lambda i, j: (i, 0)),
        ],
        out_specs=pl.BlockSpec((1, block_s, d), lambda i, j: (j, i, 0)),
        out_shape=jax.ShapeDtypeStruct((b, s, d), x.dtype),
    )(x, pe)


def kernel(x, pe):
    return _pe_add(x, pe)


# speedup vs baseline: 2.1013x; 1.0866x over previous
import functools

import jax
import jax.numpy as jnp
from jax.experimental import pallas as pl


def _add_block(x_ref, pe_ref, o_ref):
    o_ref[...] = x_ref[...] + pe_ref[...][None, :, :]


@functools.partial(jax.jit, static_argnames=("block_s",))
def _pe_add(x, pe, block_s=512):
    b, s, d = x.shape
    grid = (s // block_s, b)
    return pl.pallas_call(
        _add_block,
        grid=grid,
        in_specs=[
            pl.BlockSpec((1, block_s, d), lambda i, j: (j, i, 0)),
            pl.BlockSpec((block_s, d), lambda i, j: (i, 0)),
        ],
        out_specs=pl.BlockSpec((1, block_s, d), lambda i, j: (j, i, 0)),
        out_shape=jax.ShapeDtypeStruct((b, s, d), x.dtype),
    )(x, pe)


def kernel(x, pe):
    return _pe_add(x, pe, block_s=1024)


# speedup vs baseline: 2.2794x; 1.0847x over previous
import functools

import jax
import jax.numpy as jnp
from jax.experimental import pallas as pl


def _add_block(x_ref, pe_ref, o_ref):
    o_ref[...] = x_ref[...] + pe_ref[...][None, :, :]


@functools.partial(jax.jit, static_argnames=("block_s",))
def _pe_add(x, pe, block_s=512):
    b, s, d = x.shape
    grid = (s // block_s, b)
    return pl.pallas_call(
        _add_block,
        grid=grid,
        in_specs=[
            pl.BlockSpec((1, block_s, d), lambda i, j: (j, i, 0)),
            pl.BlockSpec((block_s, d), lambda i, j: (i, 0)),
        ],
        out_specs=pl.BlockSpec((1, block_s, d), lambda i, j: (j, i, 0)),
        out_shape=jax.ShapeDtypeStruct((b, s, d), x.dtype),
    )(x, pe)


def kernel(x, pe):
    return _pe_add(x, pe, block_s=2048)
